# Initial kernel scaffold; baseline (speedup 1.0000x reference)
import jax
import jax.numpy as jnp
from jax.experimental import pallas as pl


def kernel(x, edge_index, edge_attr, batch, rho, W1, b1, W2, b2, M1, mb1, M2, mb2, M3, mb3):
    raise NotImplementedError("write your pallas kernel here")


# speedup vs baseline: 17.7887x; 17.7887x over previous
import functools

import jax
import jax.numpy as jnp
from jax import lax
from jax.experimental import pallas as pl
from jax.experimental.pallas import tpu as pltpu
from jax.experimental.pallas import tpu_sc as plsc

_N = 10000
_E = 320000
_D = 128
_B = 64
_H = 128
_OUT = 36

_NC = 2
_NS = 16
_NW = _NC * _NS
_CH = 128
_KW = 79
_EPAD = _NW * _KW * _CH
_NP = 10240
_RPS = _NP // _NS
_RB = 1024
_NBLK = _NP // _RB

_MESH = plsc.VectorSubcoreMesh(core_axis_name="c", subcore_axis_name="s")


@functools.partial(
    pl.kernel,
    out_type=jax.ShapeDtypeStruct((_NC, _NP), jnp.float32),
    mesh=_MESH,
    scratch_types=[
        pltpu.VMEM((_KW, _CH), jnp.int32),
        pltpu.VMEM((_KW * _CH,), jnp.float32),
        pltpu.VMEM_SHARED((_NP,), jnp.float32),
    ],
)
def _deg_kernel(dstr, ewr, zn, out, dst_v, ew_v, acc):
    c = lax.axis_index("c")
    s = lax.axis_index("s")
    w = c * _NS + s
    pltpu.sync_copy(zn.at[pl.ds(s * _RPS, _RPS)], acc.at[pl.ds(s * _RPS, _RPS)])
    pltpu.sync_copy(dstr.at[w], dst_v)
    pltpu.sync_copy(ewr.at[w], ew_v)
    plsc.subcore_barrier()

    def body(j, carry):
        pltpu.sync_copy(ew_v.at[pl.ds(j * _CH, _CH)], acc.at[dst_v.at[j]], add=True)
        return carry

    lax.fori_loop(0, _KW, body, 0)
    plsc.subcore_barrier()
    pltpu.sync_copy(acc.at[pl.ds(s * _RPS, _RPS)], out.at[c, pl.ds(s * _RPS, _RPS)])


@functools.partial(
    pl.kernel,
    out_type=jax.ShapeDtypeStruct((_NC, _NP, _D), jnp.float32),
    mesh=_MESH,
    scratch_types=[
        pltpu.VMEM((_KW, _CH), jnp.int32),
        pltpu.VMEM((_KW, _CH), jnp.int32),
        pltpu.VMEM((_KW * _CH,), jnp.float32),
        pltpu.VMEM((_CH, _D), jnp.float32),
        pltpu.VMEM_SHARED((_NP, _D), jnp.float32),
        pltpu.SemaphoreType.DMA,
    ],
)
def _agg_kernel(g, srcr, dstr, ewr, znd, out, src_v, dst_v, ew_v, rows_v, acc, sem):
    c = lax.axis_index("c")
    s = lax.axis_index("s")
    w = c * _NS + s
    pltpu.sync_copy(znd.at[pl.ds(s * _RPS, _RPS)], acc.at[pl.ds(s * _RPS, _RPS)])
    pltpu.sync_copy(srcr.at[w], src_v)
    pltpu.sync_copy(dstr.at[w], dst_v)
    pltpu.sync_copy(ewr.at[w], ew_v)
    plsc.subcore_barrier()

    def chunk(j, carry):
        pltpu.async_copy(g.at[src_v.at[j]], rows_v, sem).wait()

        dn = lax.GatherDimensionNumbers(
            offset_dims=(), collapsed_slice_dims=(0,), start_index_map=(0,))

        def group(gidx, gcarry):
            ewg = ew_v[pl.ds(j * _CH + gidx * 16, 16)]
            for l in range(16):
                sp = lax.gather(ewg, jnp.full((16, 1), l, jnp.int32), dn, (1,),
                                mode=lax.GatherScatterMode.PROMISE_IN_BOUNDS)
                r = gidx * 16 + l
                for f in range(_D // 16):
                    rows_v[r, pl.ds(f * 16, 16)] = rows_v[r, pl.ds(f * 16, 16)] * sp
            return gcarry

        lax.fori_loop(0, _CH // 16, group, 0)
        pltpu.sync_copy(rows_v, acc.at[dst_v.at[j]], add=True)
        return carry

    lax.fori_loop(0, _KW, chunk, 0)
    plsc.subcore_barrier()
    pltpu.sync_copy(acc.at[pl.ds(s * _RPS, _RPS)], out.at[c, pl.ds(s * _RPS, _RPS)])


def _tc1_body(x_ref, degp_ref, w1_ref, hw_ref, g_ref):
    deg = degp_ref[0] + degp_ref[1] + 1.0
    dinv = lax.rsqrt(deg)
    hw = jnp.dot(x_ref[...], w1_ref[...], preferred_element_type=jnp.float32)
    hw_ref[...] = hw
    g_ref[...] = hw * dinv


_tc1 = pl.pallas_call(
    _tc1_body,
    grid=(_NBLK,),
    in_specs=[
        pl.BlockSpec((_RB, _D), lambda i: (i, 0)),
        pl.BlockSpec((_NC, _RB, 1), lambda i: (0, i, 0)),
        pl.BlockSpec((_D, _H), lambda i: (0, 0)),
    ],
    out_specs=[
        pl.BlockSpec((_RB, _H), lambda i: (i, 0)),
        pl.BlockSpec((_RB, _H), lambda i: (i, 0)),
    ],
    out_shape=[
        jax.ShapeDtypeStruct((_NP, _H), jnp.float32),
        jax.ShapeDtypeStruct((_NP, _H), jnp.float32),
    ],
)


def _tc2_body(aggp_ref, hw1_ref, degp_ref, w2_ref, b1_ref, hw2_ref, g2_ref):
    agg = aggp_ref[0] + aggp_ref[1]
    deg = degp_ref[0] + degp_ref[1] + 1.0
    dinv = lax.rsqrt(deg)
    h1 = jnp.maximum(dinv * agg + (dinv * dinv) * hw1_ref[...] + b1_ref[...], 0.0)
    hw2 = jnp.dot(h1, w2_ref[...], preferred_element_type=jnp.float32)
    hw2_ref[...] = hw2
    g2_ref[...] = hw2 * dinv


_tc2 = pl.pallas_call(
    _tc2_body,
    grid=(_NBLK,),
    in_specs=[
        pl.BlockSpec((_NC, _RB, _H), lambda i: (0, i, 0)),
        pl.BlockSpec((_RB, _H), lambda i: (i, 0)),
        pl.BlockSpec((_NC, _RB, 1), lambda i: (0, i, 0)),
        pl.BlockSpec((_H, _H), lambda i: (0, 0)),
        pl.BlockSpec((1, _H), lambda i: (0, 0)),
    ],
    out_specs=[
        pl.BlockSpec((_RB, _H), lambda i: (i, 0)),
        pl.BlockSpec((_RB, _H), lambda i: (i, 0)),
    ],
    out_shape=[
        jax.ShapeDtypeStruct((_NP, _H), jnp.float32),
        jax.ShapeDtypeStruct((_NP, _H), jnp.float32),
    ],
)


def _tc3_body(aggp_ref, hw2_ref, degp_ref, batch_ref, b2_ref, rho_ref,
              m1a_ref, m1b_ref, m1c_ref, mb1_ref, m2_ref, mb2_ref, m3_ref, mb3_ref,
              out_ref, ssum, smax, scnt):
    i = pl.program_id(0)

    @pl.when(i == 0)
    def _init():
        ssum[...] = jnp.zeros_like(ssum)
        smax[...] = jnp.full_like(smax, -jnp.inf)
        scnt[...] = jnp.zeros_like(scnt)

    agg = aggp_ref[0] + aggp_ref[1]
    deg = degp_ref[0] + degp_ref[1] + 1.0
    dinv = lax.rsqrt(deg)
    h2 = jnp.maximum(dinv * agg + (dinv * dinv) * hw2_ref[...] + b2_ref[...], 0.0)

    bt = batch_ref[...]
    oh = (bt == lax.broadcasted_iota(jnp.int32, (1, _B), 1)).astype(jnp.float32)
    dn = (((0,), (0,)), ((), ()))
    ssum[...] += lax.dot_general(oh, h2, dn, preferred_element_type=jnp.float32)
    scnt[...] += lax.dot_general(oh, jnp.ones((_RB, 1), jnp.float32), dn,
                                 preferred_element_type=jnp.float32)

    bmin = jnp.min(bt)
    bmax = jnp.max(bt)
    for b in range(_B):
        @pl.when((bmin <= b) & (b <= bmax))
        def _seg(b=b):
            m = bt == b
            contrib = jnp.max(jnp.where(m, h2, -jnp.inf), axis=0, keepdims=True)
            smax[b:b + 1, :] = jnp.maximum(smax[b:b + 1, :], contrib)

    @pl.when(i == pl.num_programs(0) - 1)
    def _fin():
        gmp = smax[...]
        gmp = jnp.where(gmp == -jnp.inf, 0.0, gmp)
        gap = ssum[...] / jnp.maximum(scnt[...], 1.0)
        z = (jnp.dot(gmp, m1a_ref[...], preferred_element_type=jnp.float32)
             + jnp.dot(gap, m1b_ref[...], preferred_element_type=jnp.float32)
             + jnp.dot(rho_ref[...], m1c_ref[...], preferred_element_type=jnp.float32)
             + mb1_ref[...])
        z = jnp.maximum(z, 0.0)
        z = jnp.maximum(jnp.dot(z, m2_ref[...], preferred_element_type=jnp.float32)
                        + mb2_ref[...], 0.0)
        out_ref[...] = (jnp.dot(z, m3_ref[...], preferred_element_type=jnp.float32)
                        + mb3_ref[...])


_tc3 = pl.pallas_call(
    _tc3_body,
    grid=(_NBLK,),
    in_specs=[
        pl.BlockSpec((_NC, _RB, _H), lambda i: (0, i, 0)),
        pl.BlockSpec((_RB, _H), lambda i: (i, 0)),
        pl.BlockSpec((_NC, _RB, 1), lambda i: (0, i, 0)),
        pl.BlockSpec((_RB, 1), lambda i: (i, 0)),
        pl.BlockSpec((1, _H), lambda i: (0, 0)),
        pl.BlockSpec((_B, 1), lambda i: (0, 0)),
        pl.BlockSpec((_H, _H), lambda i: (0, 0)),
        pl.BlockSpec((_H, _H), lambda i: (0, 0)),
        pl.BlockSpec((1, _H), lambda i: (0, 0)),
        pl.BlockSpec((1, _H), lambda i: (0, 0)),
        pl.BlockSpec((_H, _H), lambda i: (0, 0)),
        pl.BlockSpec((1, _H), lambda i: (0, 0)),
        pl.BlockSpec((_H, _OUT), lambda i: (0, 0)),
        pl.BlockSpec((1, _OUT), lambda i: (0, 0)),
    ],
    out_specs=pl.BlockSpec((_B, _OUT), lambda i: (0, 0)),
    out_shape=jax.ShapeDtypeStruct((_B, _OUT), jnp.float32),
    scratch_shapes=[
        pltpu.VMEM((_B, _H), jnp.float32),
        pltpu.VMEM((_B, _H), jnp.float32),
        pltpu.VMEM((_B, 1), jnp.float32),
    ],
)


def kernel(x, edge_index, edge_attr, batch, rho, W1, b1, W2, b2,
           M1, mb1, M2, mb2, M3, mb3):
    xp = jnp.pad(x, ((0, _NP - _N), (0, 0)))
    batch_p = jnp.pad(batch, (0, _NP - _N), constant_values=_B).reshape(_NP, 1)
    pad = _EPAD - _E
    apad = jnp.arange(pad, dtype=jnp.int32)
    src = jnp.concatenate([edge_index[0], apad % _N])
    dst = jnp.concatenate([edge_index[1], _N + apad % (_NP - _N)])
    ew = jnp.concatenate([edge_attr, jnp.zeros((pad,), jnp.float32)])
    srcr = src.reshape(_NW, _KW, _CH)
    dstr = dst.reshape(_NW, _KW, _CH)
    ewr = ew.reshape(_NW, _KW * _CH)
    zn = jnp.zeros((_NP,), jnp.float32)
    znd = jnp.zeros((_NP, _D), jnp.float32)
    rho_c = rho.reshape(_B, 1)
    m1a = M1[:_H]
    m1b = M1[_H:2 * _H]
    m1c = M1[2 * _H:]

    degp = _deg_kernel(dstr, ewr, zn).reshape(_NC, _NP, 1)
    hw1, g1 = _tc1(xp, degp, W1)
    aggp1 = _agg_kernel(g1, srcr, dstr, ewr, znd)
    hw2, g2 = _tc2(aggp1, hw1, degp, W2, b1.reshape(1, _H))
    aggp2 = _agg_kernel(g2, srcr, dstr, ewr, znd)
    return _tc3(aggp2, hw2, degp, batch_p, b2.reshape(1, _H), rho_c,
                m1a, m1b, m1c, mb1.reshape(1, _H), M2, mb2.reshape(1, _H),
                M3, mb3.reshape(1, _OUT))
